# baseline (speedup 1.0000x reference)
import jax
import jax.numpy as jnp
from jax import lax
from jax.experimental import pallas as pl
from jax.experimental.pallas import tpu as pltpu
from jax.experimental.pallas import tpu_sc as plsc

_SC_PARAMS = pltpu.CompilerParams(needs_layout_passes=False)

N = 100000
D = 128
G = 512
NC = 2
NS = 16
NW = NC * NS
CH = 80
NCHUNKS = N // CH
BPB = 2
CH1 = CH * BPB
NB1 = N // CH1
L = 16
DV = D // L
GSLAB = G // NS


def _zero_vmem2d(ref, rows, cols):
    z = jnp.zeros((L,), jnp.float32)
    for i in range(rows):
        for j in range(cols // L):
            ref[i, pl.ds(j * L, L)] = z


def _sc_pass1_body(x_hbm, b2_hbm, sums_out, cnt_out,
                   xb0, xb1, ix0, ix1, zbuf, hist, acc_sh,
                   sl0, sl1, ss):
    cid = lax.axis_index("c")
    sid = lax.axis_index("s")
    wid = sid * NC + cid
    slab = pl.ds(sid * GSLAB, GSLAB)

    _zero_vmem2d(zbuf, GSLAB, D)
    pltpu.sync_copy(zbuf, acc_sh.at[slab])
    zv = jnp.zeros((L,), jnp.float32)

    @plsc.parallel_loop(0, G, unroll=8)
    def _(i):
        hist[pl.ds(i * L, L)] = zv

    plsc.subcore_barrier()

    nloops = (NB1 - wid + NW - 1) // NW
    iota = lax.iota(jnp.int32, L)
    ones16 = jnp.ones((L,), jnp.float32)
    xbufs = (xb0, xb1)
    ixs = (ix0, ix1)
    semL = (sl0, sl1)

    def startL(k, p):
        blk = wid + k * NW
        pltpu.async_copy(b2_hbm.at[pl.ds(blk * BPB, BPB)], ixs[p], semL[p])
        pltpu.async_copy(x_hbm.at[pl.ds(blk * CH1, CH1)], xbufs[p], semL[p])

    def waitL(k, p):
        blk = wid + k * NW
        pltpu.make_async_copy(
            b2_hbm.at[pl.ds(blk * BPB, BPB)], ixs[p], semL[p]).wait()
        pltpu.make_async_copy(
            x_hbm.at[pl.ds(blk * CH1, CH1)], xbufs[p], semL[p]).wait()

    def process(p):
        xbuf, idx2 = xbufs[p], ixs[p]
        d0 = pltpu.async_copy(xbuf.at[pl.ds(0, CH)],
                              acc_sh.at[idx2.at[0]], ss, add=True)
        d1 = pltpu.async_copy(xbuf.at[pl.ds(CH, CH)],
                              acc_sh.at[idx2.at[1]], ss, add=True)
        for r in range(BPB):
            for o in range(CH // L):
                a = idx2[r, pl.ds(o * L, L)] * L + iota
                plsc.addupdate_scatter(hist, [a], ones16)
        d0.wait()
        d1.wait()

    startL(0, 0)

    @pl.when(nloops > 1)
    def _():
        startL(1, 1)

    def body(k2, carry):
        cA = 2 * k2
        cB = cA + 1
        waitL(cA, 0)
        process(0)

        @pl.when(cA + 2 < nloops)
        def _():
            startL(cA + 2, 0)

        @pl.when(cB < nloops)
        def _():
            waitL(cB, 1)
            process(1)

            @pl.when(cB + 2 < nloops)
            def _():
                startL(cB + 2, 1)

        return carry

    lax.fori_loop(0, (nloops + 1) // 2, body, 0)
    plsc.subcore_barrier()

    pltpu.sync_copy(acc_sh.at[slab], zbuf)
    pltpu.sync_copy(zbuf, sums_out.at[cid, slab])
    pltpu.sync_copy(hist, cnt_out.at[cid, sid])


def _sc_pass1(x, batch2):
    mesh = plsc.VectorSubcoreMesh(core_axis_name="c", subcore_axis_name="s")
    return pl.kernel(
        _sc_pass1_body,
        out_type=(
            jax.ShapeDtypeStruct((NC, G, D), jnp.float32),
            jax.ShapeDtypeStruct((NC, NS, G * L), jnp.float32),
        ),
        mesh=mesh,
        scratch_types=[
            pltpu.VMEM((CH1, D), jnp.float32),
            pltpu.VMEM((CH1, D), jnp.float32),
            pltpu.VMEM((BPB, CH), jnp.int32),
            pltpu.VMEM((BPB, CH), jnp.int32),
            pltpu.VMEM((GSLAB, D), jnp.float32),
            pltpu.VMEM((G * L,), jnp.float32),
            pltpu.VMEM_SHARED((G, D), jnp.float32),
            pltpu.SemaphoreType.DMA,
            pltpu.SemaphoreType.DMA,
            pltpu.SemaphoreType.DMA,
        ],
        compiler_params=_SC_PARAMS,
    )(x, batch2)


def _tc_gate_body(sums_ref, cnt_ref, w_ref, tg_ref):
    sums = sums_ref[0] + sums_ref[1]
    hists = cnt_ref[...].reshape(NC * NS, G, L)
    counts = jnp.sum(hists, axis=(0, 2))
    mean = sums / jnp.maximum(counts, 1.0)[:, None]
    tg_ref[...] = jnp.tanh(
        jnp.dot(mean, w_ref[...], preferred_element_type=jnp.float32))


def _tc_gate(sums_p, cnt_p, W):
    return pl.pallas_call(
        _tc_gate_body,
        out_shape=jax.ShapeDtypeStruct((G, D), jnp.float32),
    )(sums_p, cnt_p, W)


def _sc_pass2_body(x_hbm, b2_hbm, tg_hbm, out_hbm,
                   xb0, xb1, ob0, ob1, tr0, tr1, iv0, iv1, iv1a, iv1b,
                   zbuf, tg_sh, acc_sh,
                   sl0, sl1, sg0, sg1):
    cid = lax.axis_index("c")
    sid = lax.axis_index("s")
    wid = sid * NC + cid
    slab = pl.ds(sid * GSLAB, GSLAB)

    pltpu.sync_copy(tg_hbm.at[slab], zbuf)
    pltpu.sync_copy(zbuf, tg_sh.at[slab])
    _zero_vmem2d(zbuf, GSLAB, D)
    pltpu.sync_copy(zbuf, acc_sh.at[slab])
    plsc.subcore_barrier()

    nloops = (NCHUNKS - wid + NW - 1) // NW
    xbufs = (xb0, xb1)
    obufs = (ob0, ob1)
    tgs = (tr0, tr1)
    ivs = (iv0, iv1)
    iv1s = (iv1a, iv1b)
    semL = (sl0, sl1)
    semG = (sg0, sg1)

    def startL(k, p):
        c = wid + k * NW
        pltpu.async_copy(b2_hbm.at[c], ivs[p], semL[p])
        pltpu.async_copy(b2_hbm.at[c, pl.ds(0, 1)], iv1s[p], semL[p])
        pltpu.async_copy(x_hbm.at[pl.ds(c * CH, CH)], xbufs[p], semL[p])

    def waitL(k, p):
        c = wid + k * NW
        pltpu.make_async_copy(b2_hbm.at[c], ivs[p], semL[p]).wait()
        pltpu.make_async_copy(b2_hbm.at[c, pl.ds(0, 1)], iv1s[p],
                              semL[p]).wait()
        pltpu.make_async_copy(
            x_hbm.at[pl.ds(c * CH, CH)], xbufs[p], semL[p]).wait()

    def startG(p):
        pltpu.async_copy(tg_sh.at[ivs[p]], tgs[p], semG[p])

    def waitG(p):
        pltpu.make_async_copy(tg_sh.at[ivs[p]], tgs[p], semG[p]).wait()

    def process(p):
        xbuf, obuf, tgrows, idxv = xbufs[p], obufs[p], tgs[p], ivs[p]
        waitG(p)
        v0 = idxv[pl.ds(0, L)]
        v4 = idxv[pl.ds(4 * L, L)]
        uniform = jnp.sum(v4 - v0) == 0

        @pl.when(uniform)
        def _():
            tr = [tgrows[0, pl.ds(j * L, L)] for j in range(DV)]
            zero8 = tuple(jnp.zeros((L,), jnp.float32) for _ in range(DV))

            @plsc.parallel_loop(0, CH, unroll=16, carry=zero8)
            def rowf(i, acc):
                xr = [xbuf[i, pl.ds(j * L, L)] for j in range(DV)]
                pr = [xr[j] * tr[j] for j in range(DV)]
                s = ((pr[0] + pr[1]) + (pr[2] + pr[3])) + (
                    (pr[4] + pr[5]) + (pr[6] + pr[7]))
                tot = jnp.sum(s)
                coef = 1.0 / (1.0 + jnp.exp(jnp.broadcast_to(-tot, (L,))))
                return tuple(acc[j] + xr[j] * coef for j in range(DV))

            for j in range(DV):
                obuf[0, pl.ds(j * L, L)] = rowf[j]
            pltpu.sync_copy(obuf.at[pl.ds(0, 1)], acc_sh.at[iv1s[p]],
                            add=True)

        @pl.when(jnp.logical_not(uniform))
        def _():
            @plsc.parallel_loop(0, CH, unroll=16)
            def _(i):
                xr = [xbuf[i, pl.ds(j * L, L)] for j in range(DV)]
                pr = [xr[j] * tgrows[i, pl.ds(j * L, L)] for j in range(DV)]
                s = ((pr[0] + pr[1]) + (pr[2] + pr[3])) + (
                    (pr[4] + pr[5]) + (pr[6] + pr[7]))
                tot = jnp.sum(s)
                coef = 1.0 / (1.0 + jnp.exp(jnp.broadcast_to(-tot, (L,))))
                for j in range(DV):
                    obuf[i, pl.ds(j * L, L)] = xr[j] * coef

            pltpu.sync_copy(obuf, acc_sh.at[idxv], add=True)

    startL(0, 0)

    @pl.when(nloops > 1)
    def _():
        startL(1, 1)

    waitL(0, 0)
    startG(0)

    def body(k2, carry):
        cA = 2 * k2
        cB = cA + 1

        @pl.when(cB < nloops)
        def _():
            waitL(cB, 1)
            startG(1)

        process(0)

        @pl.when(cA + 2 < nloops)
        def _():
            startL(cA + 2, 0)

        @pl.when(cB < nloops)
        def _():
            process(1)

            @pl.when(cA + 2 < nloops)
            def _():
                waitL(cA + 2, 0)
                startG(0)

            @pl.when(cB + 2 < nloops)
            def _():
                startL(cB + 2, 1)

        return carry

    lax.fori_loop(0, (nloops + 1) // 2, body, 0)
    plsc.subcore_barrier()

    pltpu.sync_copy(acc_sh.at[slab], zbuf)
    pltpu.sync_copy(zbuf, out_hbm.at[cid, slab])


def _sc_pass2(x, batch2, tg):
    mesh = plsc.VectorSubcoreMesh(core_axis_name="c", subcore_axis_name="s")
    return pl.kernel(
        _sc_pass2_body,
        out_type=jax.ShapeDtypeStruct((NC, G, D), jnp.float32),
        mesh=mesh,
        scratch_types=[
            pltpu.VMEM((CH, D), jnp.float32),
            pltpu.VMEM((CH, D), jnp.float32),
            pltpu.VMEM((CH, D), jnp.float32),
            pltpu.VMEM((CH, D), jnp.float32),
            pltpu.VMEM((CH, D), jnp.float32),
            pltpu.VMEM((CH, D), jnp.float32),
            pltpu.VMEM((CH,), jnp.int32),
            pltpu.VMEM((CH,), jnp.int32),
            pltpu.VMEM((1,), jnp.int32),
            pltpu.VMEM((1,), jnp.int32),
            pltpu.VMEM((GSLAB, D), jnp.float32),
            pltpu.VMEM_SHARED((G, D), jnp.float32),
            pltpu.VMEM_SHARED((G, D), jnp.float32),
            pltpu.SemaphoreType.DMA,
            pltpu.SemaphoreType.DMA,
            pltpu.SemaphoreType.DMA,
            pltpu.SemaphoreType.DMA,
        ],
        compiler_params=_SC_PARAMS,
    )(x, batch2, tg)


def _tc_add_body(p_ref, out_ref):
    out_ref[...] = p_ref[0] + p_ref[1]


def _tc_add(out_p):
    return pl.pallas_call(
        _tc_add_body,
        out_shape=jax.ShapeDtypeStruct((G, D), jnp.float32),
    )(out_p)


@jax.jit
def kernel(x, batch, W):
    batch2 = batch.astype(jnp.int32).reshape(NCHUNKS, CH)
    sums_p, cnt_p = _sc_pass1(x, batch2)
    tg = _tc_gate(sums_p, cnt_p, W)
    out_p = _sc_pass2(x, batch2, tg)
    return _tc_add(out_p)


# speedup vs baseline: 1.3318x; 1.3318x over previous
import jax
import jax.numpy as jnp
from jax import lax
from jax.experimental import pallas as pl
from jax.experimental.pallas import tpu as pltpu
from jax.experimental.pallas import tpu_sc as plsc

_SC_PARAMS = pltpu.CompilerParams(needs_layout_passes=False)

N = 100000
D = 128
G = 512
NC = 2
NS = 16
NW = NC * NS
CH = 80
NCHUNKS = N // CH
BPB = 2
CH1 = CH * BPB
NB1 = N // CH1
L = 16
DV = D // L
GSLAB = G // NS


def _zero_vmem2d(ref, rows, cols):
    z = jnp.zeros((L,), jnp.float32)
    for i in range(rows):
        for j in range(cols // L):
            ref[i, pl.ds(j * L, L)] = z


def _sc_pass1_body(x_hbm, b2_hbm, sums_out, cnt_out,
                   xb0, xb1, ix0, ix1, zbuf, hist, acc_sh,
                   sl0, sl1, ss):
    cid = lax.axis_index("c")
    sid = lax.axis_index("s")
    wid = sid * NC + cid
    slab = pl.ds(sid * GSLAB, GSLAB)

    _zero_vmem2d(zbuf, GSLAB, D)
    pltpu.sync_copy(zbuf, acc_sh.at[slab])
    zv = jnp.zeros((L,), jnp.float32)

    @plsc.parallel_loop(0, G, unroll=8)
    def _(i):
        hist[pl.ds(i * L, L)] = zv

    plsc.subcore_barrier()

    nloops = (NB1 - wid + NW - 1) // NW
    iota = lax.iota(jnp.int32, L)
    ones16 = jnp.ones((L,), jnp.float32)
    xbufs = (xb0, xb1)
    ixs = (ix0, ix1)
    semL = (sl0, sl1)

    def startL(k, p):
        blk = wid + k * NW
        pltpu.async_copy(b2_hbm.at[pl.ds(blk * BPB, BPB)], ixs[p], semL[p])
        pltpu.async_copy(x_hbm.at[pl.ds(blk * CH1, CH1)], xbufs[p], semL[p])

    def waitL(k, p):
        blk = wid + k * NW
        pltpu.make_async_copy(
            b2_hbm.at[pl.ds(blk * BPB, BPB)], ixs[p], semL[p]).wait()
        pltpu.make_async_copy(
            x_hbm.at[pl.ds(blk * CH1, CH1)], xbufs[p], semL[p]).wait()

    def process(p):
        xbuf, idx2 = xbufs[p], ixs[p]
        d0 = pltpu.async_copy(xbuf.at[pl.ds(0, CH)],
                              acc_sh.at[idx2.at[0]], ss, add=True)
        d1 = pltpu.async_copy(xbuf.at[pl.ds(CH, CH)],
                              acc_sh.at[idx2.at[1]], ss, add=True)
        for r in range(BPB):
            for o in range(CH // L):
                a = idx2[r, pl.ds(o * L, L)] * L + iota
                plsc.addupdate_scatter(hist, [a], ones16)
        d0.wait()
        d1.wait()

    startL(0, 0)

    @pl.when(nloops > 1)
    def _():
        startL(1, 1)

    def body(k2, carry):
        cA = 2 * k2
        cB = cA + 1
        waitL(cA, 0)
        process(0)

        @pl.when(cA + 2 < nloops)
        def _():
            startL(cA + 2, 0)

        @pl.when(cB < nloops)
        def _():
            waitL(cB, 1)
            process(1)

            @pl.when(cB + 2 < nloops)
            def _():
                startL(cB + 2, 1)

        return carry

    lax.fori_loop(0, (nloops + 1) // 2, body, 0)
    plsc.subcore_barrier()

    pltpu.sync_copy(acc_sh.at[slab], zbuf)
    pltpu.sync_copy(zbuf, sums_out.at[cid, slab])
    pltpu.sync_copy(hist, cnt_out.at[cid, sid])


def _sc_pass1(x, batch2):
    mesh = plsc.VectorSubcoreMesh(core_axis_name="c", subcore_axis_name="s")
    return pl.kernel(
        _sc_pass1_body,
        out_type=(
            jax.ShapeDtypeStruct((NC, G, D), jnp.float32),
            jax.ShapeDtypeStruct((NC, NS, G * L), jnp.float32),
        ),
        mesh=mesh,
        scratch_types=[
            pltpu.VMEM((CH1, D), jnp.float32),
            pltpu.VMEM((CH1, D), jnp.float32),
            pltpu.VMEM((BPB, CH), jnp.int32),
            pltpu.VMEM((BPB, CH), jnp.int32),
            pltpu.VMEM((GSLAB, D), jnp.float32),
            pltpu.VMEM((G * L,), jnp.float32),
            pltpu.VMEM_SHARED((G, D), jnp.float32),
            pltpu.SemaphoreType.DMA,
            pltpu.SemaphoreType.DMA,
            pltpu.SemaphoreType.DMA,
        ],
        compiler_params=_SC_PARAMS,
    )(x, batch2)


def _tc_gate_body(sums_ref, cnt_ref, w_ref, tg_ref):
    sums = sums_ref[0] + sums_ref[1]
    hists = cnt_ref[...].reshape(NC * NS, G, L)
    counts = jnp.sum(hists, axis=(0, 2))
    mean = sums / jnp.maximum(counts, 1.0)[:, None]
    tg_ref[...] = jnp.tanh(
        jnp.dot(mean, w_ref[...], preferred_element_type=jnp.float32))


def _tc_gate(sums_p, cnt_p, W):
    return pl.pallas_call(
        _tc_gate_body,
        out_shape=jax.ShapeDtypeStruct((G, D), jnp.float32),
    )(sums_p, cnt_p, W)


def _sc_pass2_body(x_hbm, b2_hbm, tg_hbm, out_hbm,
                   xb0, xb1, ob0, ob1, tr0, tr1, iv0, iv1, iv1a, iv1b,
                   zbuf, tg_sh, acc_sh,
                   sl0, sl1, sg0, sg1):
    cid = lax.axis_index("c")
    sid = lax.axis_index("s")
    wid = sid * NC + cid
    slab = pl.ds(sid * GSLAB, GSLAB)

    pltpu.sync_copy(tg_hbm.at[slab], zbuf)
    pltpu.sync_copy(zbuf, tg_sh.at[slab])
    _zero_vmem2d(zbuf, GSLAB, D)
    pltpu.sync_copy(zbuf, acc_sh.at[slab])
    plsc.subcore_barrier()

    nloops = (NCHUNKS - wid + NW - 1) // NW
    xbufs = (xb0, xb1)
    obufs = (ob0, ob1)
    tgs = (tr0, tr1)
    ivs = (iv0, iv1)
    iv1s = (iv1a, iv1b)
    semL = (sl0, sl1)
    semG = (sg0, sg1)

    def startL(k, p):
        c = wid + k * NW
        pltpu.async_copy(b2_hbm.at[c], ivs[p], semL[p])
        pltpu.async_copy(b2_hbm.at[c, pl.ds(0, 1)], iv1s[p], semL[p])
        pltpu.async_copy(x_hbm.at[pl.ds(c * CH, CH)], xbufs[p], semL[p])

    def waitL(k, p):
        c = wid + k * NW
        pltpu.make_async_copy(b2_hbm.at[c], ivs[p], semL[p]).wait()
        pltpu.make_async_copy(b2_hbm.at[c, pl.ds(0, 1)], iv1s[p],
                              semL[p]).wait()
        pltpu.make_async_copy(
            x_hbm.at[pl.ds(c * CH, CH)], xbufs[p], semL[p]).wait()

    def startG(p):
        pltpu.async_copy(tg_sh.at[ivs[p]], tgs[p], semG[p])

    def waitG(p):
        pltpu.make_async_copy(tg_sh.at[ivs[p]], tgs[p], semG[p]).wait()

    def process(p):
        xbuf, obuf, tgrows, idxv = xbufs[p], obufs[p], tgs[p], ivs[p]
        waitG(p)
        v0 = idxv[pl.ds(0, L)]
        v4 = idxv[pl.ds(4 * L, L)]
        uniform = jnp.sum(v4 - v0) == 0

        @pl.when(uniform)
        def _():
            tr = [tgrows[0, pl.ds(j * L, L)] for j in range(DV)]
            zero8 = tuple(jnp.zeros((L,), jnp.float32) for _ in range(DV))

            @plsc.parallel_loop(0, CH, unroll=4, carry=zero8)
            def rowf(i, acc):
                xr = [xbuf[i, pl.ds(j * L, L)] for j in range(DV)]
                pr = [xr[j] * tr[j] for j in range(DV)]
                s = ((pr[0] + pr[1]) + (pr[2] + pr[3])) + (
                    (pr[4] + pr[5]) + (pr[6] + pr[7]))
                tot = jnp.sum(s)
                coef = 1.0 / (1.0 + jnp.exp(jnp.broadcast_to(-tot, (L,))))
                return tuple(acc[j] + xr[j] * coef for j in range(DV))

            for j in range(DV):
                obuf[0, pl.ds(j * L, L)] = rowf[j]
            pltpu.sync_copy(obuf.at[pl.ds(0, 1)], acc_sh.at[iv1s[p]],
                            add=True)

        @pl.when(jnp.logical_not(uniform))
        def _():
            @plsc.parallel_loop(0, CH, unroll=4)
            def _(i):
                xr = [xbuf[i, pl.ds(j * L, L)] for j in range(DV)]
                pr = [xr[j] * tgrows[i, pl.ds(j * L, L)] for j in range(DV)]
                s = ((pr[0] + pr[1]) + (pr[2] + pr[3])) + (
                    (pr[4] + pr[5]) + (pr[6] + pr[7]))
                tot = jnp.sum(s)
                coef = 1.0 / (1.0 + jnp.exp(jnp.broadcast_to(-tot, (L,))))
                for j in range(DV):
                    obuf[i, pl.ds(j * L, L)] = xr[j] * coef

            pltpu.sync_copy(obuf, acc_sh.at[idxv], add=True)

    startL(0, 0)

    @pl.when(nloops > 1)
    def _():
        startL(1, 1)

    waitL(0, 0)
    startG(0)

    def body(k2, carry):
        cA = 2 * k2
        cB = cA + 1

        @pl.when(cB < nloops)
        def _():
            waitL(cB, 1)
            startG(1)

        process(0)

        @pl.when(cA + 2 < nloops)
        def _():
            startL(cA + 2, 0)

        @pl.when(cB < nloops)
        def _():
            process(1)

            @pl.when(cA + 2 < nloops)
            def _():
                waitL(cA + 2, 0)
                startG(0)

            @pl.when(cB + 2 < nloops)
            def _():
                startL(cB + 2, 1)

        return carry

    lax.fori_loop(0, (nloops + 1) // 2, body, 0)
    plsc.subcore_barrier()

    pltpu.sync_copy(acc_sh.at[slab], zbuf)
    pltpu.sync_copy(zbuf, out_hbm.at[cid, slab])


def _sc_pass2(x, batch2, tg):
    mesh = plsc.VectorSubcoreMesh(core_axis_name="c", subcore_axis_name="s")
    return pl.kernel(
        _sc_pass2_body,
        out_type=jax.ShapeDtypeStruct((NC, G, D), jnp.float32),
        mesh=mesh,
        scratch_types=[
            pltpu.VMEM((CH, D), jnp.float32),
            pltpu.VMEM((CH, D), jnp.float32),
            pltpu.VMEM((CH, D), jnp.float32),
            pltpu.VMEM((CH, D), jnp.float32),
            pltpu.VMEM((CH, D), jnp.float32),
            pltpu.VMEM((CH, D), jnp.float32),
            pltpu.VMEM((CH,), jnp.int32),
            pltpu.VMEM((CH,), jnp.int32),
            pltpu.VMEM((1,), jnp.int32),
            pltpu.VMEM((1,), jnp.int32),
            pltpu.VMEM((GSLAB, D), jnp.float32),
            pltpu.VMEM_SHARED((G, D), jnp.float32),
            pltpu.VMEM_SHARED((G, D), jnp.float32),
            pltpu.SemaphoreType.DMA,
            pltpu.SemaphoreType.DMA,
            pltpu.SemaphoreType.DMA,
            pltpu.SemaphoreType.DMA,
        ],
        compiler_params=_SC_PARAMS,
    )(x, batch2, tg)


def _tc_add_body(p_ref, out_ref):
    out_ref[...] = p_ref[0] + p_ref[1]


def _tc_add(out_p):
    return pl.pallas_call(
        _tc_add_body,
        out_shape=jax.ShapeDtypeStruct((G, D), jnp.float32),
    )(out_p)


@jax.jit
def kernel(x, batch, W):
    batch2 = batch.astype(jnp.int32).reshape(NCHUNKS, CH)
    sums_p, cnt_p = _sc_pass1(x, batch2)
    tg = _tc_gate(sums_p, cnt_p, W)
    out_p = _sc_pass2(x, batch2, tg)
    return _tc_add(out_p)
